# baseline (speedup 1.0000x reference)
import functools

import jax
import jax.numpy as jnp
from jax import lax
from jax.experimental import pallas as pl
from jax.experimental.pallas import tpu as pltpu
from jax.experimental.pallas import tpu_sc as plsc

N_USERS = 1000000
N_ITEMS = 1000000
EMBED_DIM = 16
N_CLIMATE = 4
BATCH = 16384

NUM_CORES = 2
NUM_SUBCORES = 16
LANES = 16
NW = NUM_CORES * NUM_SUBCORES
B_PER_W = BATCH // NW
CHUNKS = B_PER_W // LANES
HALF = LANES // 2


@functools.partial(
    pl.kernel,
    out_type=jax.ShapeDtypeStruct((BATCH,), jnp.float32),
    mesh=plsc.VectorSubcoreMesh(core_axis_name="c", subcore_axis_name="s"),
    compiler_params=pltpu.CompilerParams(
        needs_layout_passes=False, use_tc_tiling_on_sc=False),
    scratch_types=[
        pltpu.VMEM((B_PER_W,), jnp.int32),
        pltpu.VMEM((B_PER_W,), jnp.int32),
        pltpu.VMEM((B_PER_W,), jnp.float32),
        pltpu.VMEM((B_PER_W,), jnp.float32),
        pltpu.VMEM((N_CLIMATE * B_PER_W,), jnp.float32),
        pltpu.VMEM((5 * LANES,), jnp.float32),
        pltpu.VMEM((B_PER_W,), jnp.float32),
        pltpu.SemaphoreType.DMA,
    ],
)
def _bias_kernel(uids_hbm, iids_hbm, ubiasT_hbm, ibiasT_hbm, clim_hbm,
                 params_hbm, out_hbm,
                 uid_v, iid_v, ub_v, ib_v, clim_v, params_v, out_v, sem):
    wid = lax.axis_index("s") * NUM_CORES + lax.axis_index("c")
    base = wid * B_PER_W

    pltpu.sync_copy(uids_hbm.at[pl.ds(base, B_PER_W)], uid_v)
    pltpu.sync_copy(iids_hbm.at[pl.ds(base, B_PER_W)], iid_v)
    pltpu.sync_copy(clim_hbm.at[pl.ds(base * N_CLIMATE, B_PER_W * N_CLIMATE)],
                    clim_v)
    pltpu.sync_copy(params_hbm, params_v)

    ub1 = ubiasT_hbm.at[0]
    ib1 = ibiasT_hbm.at[0]
    copies = []
    for j in range(B_PER_W // 128):
        sl = pl.ds(j * 128, 128)
        copies.append(pltpu.async_copy(ub1.at[uid_v.at[sl]], ub_v.at[sl], sem))
        copies.append(pltpu.async_copy(ib1.at[iid_v.at[sl]], ib_v.at[sl], sem))
    for cp in copies:
        cp.wait()

    iota = jnp.arange(LANES, dtype=jnp.int32)

    def chunk_body(c, carry):
        rows = c * LANES + iota
        acc = params_v[pl.ds(4 * LANES, LANES)]
        for cc in range(N_CLIMATE):
            f = plsc.load_gather(clim_v, [rows * N_CLIMATE + cc])
            acc = acc + f * params_v[pl.ds(cc * LANES, LANES)]
        ub = plsc.load_gather(ub_v, [rows])
        ib = plsc.load_gather(ib_v, [rows])
        out_v[pl.ds(c * LANES, LANES)] = acc + ub + ib
        return carry

    lax.fori_loop(0, CHUNKS, chunk_body, 0)
    pltpu.sync_copy(out_v, out_hbm.at[pl.ds(base, B_PER_W)])


@functools.partial(
    pl.kernel,
    out_type=jax.ShapeDtypeStruct((BATCH,), jnp.float32),
    mesh=plsc.VectorSubcoreMesh(core_axis_name="c", subcore_axis_name="s"),
    compiler_params=pltpu.CompilerParams(
        needs_layout_passes=False, use_tc_tiling_on_sc=True),
    scratch_types=[
        pltpu.VMEM((B_PER_W,), jnp.int32),
        pltpu.VMEM((B_PER_W,), jnp.int32),
        pltpu.VMEM((2, HALF, EMBED_DIM, 128), jnp.float32),
        pltpu.VMEM((2, HALF, EMBED_DIM, 128), jnp.float32),
        pltpu.VMEM((LANES, LANES), jnp.float32),
        pltpu.VMEM((B_PER_W,), jnp.float32),
        pltpu.SemaphoreType.DMA,
        pltpu.SemaphoreType.DMA,
    ],
)
def _fm_kernel(uids_hbm, iids_hbm, uembT_hbm, iembT_hbm, out_hbm,
               uid_v, iid_v, ublk_v, iblk_v, pmat_v, out_v, sem0, sem1):
    wid = lax.axis_index("s") * NUM_CORES + lax.axis_index("c")
    base = wid * B_PER_W

    pltpu.sync_copy(uids_hbm.at[pl.ds(base, B_PER_W)], uid_v)
    pltpu.sync_copy(iids_hbm.at[pl.ds(base, B_PER_W)], iid_v)

    iota = jnp.arange(LANES, dtype=jnp.int32)

    def fire_half(c, half, sem):
        ids_u = uid_v[pl.ds(c * LANES, LANES)]
        ids_i = iid_v[pl.ds(c * LANES, LANES)]
        for e in range(HALF):
            k = half * HALF + e
            ucol0 = pl.multiple_of((ids_u[k] // 128) * 128, 128)
            icol0 = pl.multiple_of((ids_i[k] // 128) * 128, 128)
            pltpu.async_copy(uembT_hbm.at[:, pl.ds(ucol0, 128)],
                             ublk_v.at[half, e], sem)
            pltpu.async_copy(iembT_hbm.at[:, pl.ds(icol0, 128)],
                             iblk_v.at[half, e], sem)

    def drain_half(sem):
        for _ in range(2 * HALF):
            pltpu.make_async_copy(
                uembT_hbm.at[:, pl.ds(0, 128)], ublk_v.at[0, 0], sem).wait()

    def compute_half(c, half):
        ids_u = uid_v[pl.ds(c * LANES, LANES)]
        ids_i = iid_v[pl.ds(c * LANES, LANES)]
        for e in range(HALF):
            k = half * HALF + e
            ulane = jnp.full((LANES,), ids_u[k] % 128, jnp.int32)
            ilane = jnp.full((LANES,), ids_i[k] % 128, jnp.int32)
            u = plsc.load_gather(ublk_v.at[half, e], [iota, ulane])
            v = plsc.load_gather(iblk_v.at[half, e], [iota, ilane])
            plsc.store_scatter(pmat_v,
                               [iota, jnp.full((LANES,), k, jnp.int32)],
                               u * v)

    fire_half(0, 0, sem0)
    fire_half(0, 1, sem1)

    def chunk_body(c, carry):
        drain_half(sem0)
        compute_half(c, 0)

        @pl.when(c + 1 < CHUNKS)
        def _():
            fire_half(c + 1, 0, sem0)

        drain_half(sem1)
        compute_half(c, 1)

        @pl.when(c + 1 < CHUNKS)
        def _():
            fire_half(c + 1, 1, sem1)

        dot = pmat_v[0]
        for r in range(1, LANES):
            dot = dot + pmat_v[r]
        out_v[pl.ds(c * LANES, LANES)] = dot
        return carry

    lax.fori_loop(0, CHUNKS, chunk_body, 0)

    pltpu.sync_copy(out_v, out_hbm.at[pl.ds(base, B_PER_W)])


def kernel(user_ids, item_ids, climate_feats, user_emb, item_emb,
           user_bias, item_bias, W_climate, b_climate, global_bias):
    w_bcast = jnp.broadcast_to(W_climate.reshape(N_CLIMATE, 1), (N_CLIMATE, LANES))
    const_bcast = jnp.broadcast_to(b_climate + global_bias, (1, LANES))
    params = jnp.concatenate([w_bcast, const_bcast], axis=0).reshape(5 * LANES)
    uids32 = user_ids.astype(jnp.int32)
    iids32 = item_ids.astype(jnp.int32)
    fm = _fm_kernel(uids32, iids32, user_emb.T, item_emb.T)
    bias = _bias_kernel(uids32, iids32, user_bias.T, item_bias.T,
                        climate_feats.reshape(BATCH * N_CLIMATE), params)
    return (fm + bias).reshape(BATCH, 1)


# speedup vs baseline: 1.6114x; 1.6114x over previous
import functools

import jax
import jax.numpy as jnp
from jax import lax
from jax.experimental import pallas as pl
from jax.experimental.pallas import tpu as pltpu
from jax.experimental.pallas import tpu_sc as plsc

N_USERS = 1000000
N_ITEMS = 1000000
EMBED_DIM = 16
N_CLIMATE = 4
BATCH = 16384

NUM_CORES = 2
NUM_SUBCORES = 16
LANES = 16
NW = NUM_CORES * NUM_SUBCORES
B_PER_W = BATCH // NW
CHUNKS = B_PER_W // LANES
HALF = LANES // 2


@functools.partial(
    pl.kernel,
    out_type=jax.ShapeDtypeStruct((BATCH,), jnp.float32),
    mesh=plsc.VectorSubcoreMesh(core_axis_name="c", subcore_axis_name="s"),
    compiler_params=pltpu.CompilerParams(
        needs_layout_passes=False, use_tc_tiling_on_sc=False),
    scratch_types=[
        pltpu.VMEM((B_PER_W,), jnp.int32),
        pltpu.VMEM((B_PER_W,), jnp.int32),
        pltpu.VMEM((B_PER_W,), jnp.float32),
        pltpu.VMEM((B_PER_W,), jnp.float32),
        pltpu.VMEM((N_CLIMATE * B_PER_W,), jnp.float32),
        pltpu.VMEM((5 * LANES,), jnp.float32),
        pltpu.VMEM((B_PER_W,), jnp.float32),
        pltpu.VMEM((B_PER_W,), jnp.float32),
        pltpu.SemaphoreType.DMA,
    ],
)
def _bias_kernel(uids_hbm, iids_hbm, ubiasT_hbm, ibiasT_hbm, clim_hbm,
                 params_hbm, fm_hbm, out_hbm,
                 uid_v, iid_v, ub_v, ib_v, clim_v, params_v, fm_v, out_v, sem):
    wid = lax.axis_index("s") * NUM_CORES + lax.axis_index("c")
    base = wid * B_PER_W

    pltpu.sync_copy(uids_hbm.at[pl.ds(base, B_PER_W)], uid_v)
    pltpu.sync_copy(iids_hbm.at[pl.ds(base, B_PER_W)], iid_v)
    pltpu.sync_copy(fm_hbm.at[pl.ds(base, B_PER_W)], fm_v)
    pltpu.sync_copy(clim_hbm.at[pl.ds(base * N_CLIMATE, B_PER_W * N_CLIMATE)],
                    clim_v)
    pltpu.sync_copy(params_hbm, params_v)

    ub1 = ubiasT_hbm.at[0]
    ib1 = ibiasT_hbm.at[0]
    copies = []
    for j in range(B_PER_W // 128):
        sl = pl.ds(j * 128, 128)
        copies.append(pltpu.async_copy(ub1.at[uid_v.at[sl]], ub_v.at[sl], sem))
        copies.append(pltpu.async_copy(ib1.at[iid_v.at[sl]], ib_v.at[sl], sem))
    for cp in copies:
        cp.wait()

    iota = jnp.arange(LANES, dtype=jnp.int32)

    def chunk_body(c, carry):
        rows = c * LANES + iota
        acc = params_v[pl.ds(4 * LANES, LANES)]
        for cc in range(N_CLIMATE):
            f = plsc.load_gather(clim_v, [rows * N_CLIMATE + cc])
            acc = acc + f * params_v[pl.ds(cc * LANES, LANES)]
        ub = plsc.load_gather(ub_v, [rows])
        ib = plsc.load_gather(ib_v, [rows])
        fm = fm_v[pl.ds(c * LANES, LANES)]
        out_v[pl.ds(c * LANES, LANES)] = acc + ub + ib + fm
        return carry

    lax.fori_loop(0, CHUNKS, chunk_body, 0)
    pltpu.sync_copy(out_v, out_hbm.at[pl.ds(base, B_PER_W)])


@functools.partial(
    pl.kernel,
    out_type=jax.ShapeDtypeStruct((BATCH,), jnp.float32),
    mesh=plsc.VectorSubcoreMesh(core_axis_name="c", subcore_axis_name="s"),
    compiler_params=pltpu.CompilerParams(
        needs_layout_passes=False, use_tc_tiling_on_sc=True),
    scratch_types=[
        pltpu.VMEM((B_PER_W,), jnp.int32),
        pltpu.VMEM((B_PER_W,), jnp.int32),
        pltpu.VMEM((2, HALF, EMBED_DIM, 128), jnp.float32),
        pltpu.VMEM((2, HALF, EMBED_DIM, 128), jnp.float32),
        pltpu.VMEM((LANES, LANES), jnp.float32),
        pltpu.VMEM((B_PER_W,), jnp.float32),
        pltpu.SemaphoreType.DMA,
        pltpu.SemaphoreType.DMA,
    ],
)
def _fm_kernel(uids_hbm, iids_hbm, uembT_hbm, iembT_hbm, out_hbm,
               uid_v, iid_v, ublk_v, iblk_v, pmat_v, out_v, sem0, sem1):
    wid = lax.axis_index("s") * NUM_CORES + lax.axis_index("c")
    base = wid * B_PER_W

    pltpu.sync_copy(uids_hbm.at[pl.ds(base, B_PER_W)], uid_v)
    pltpu.sync_copy(iids_hbm.at[pl.ds(base, B_PER_W)], iid_v)

    iota = jnp.arange(LANES, dtype=jnp.int32)

    def fire_half(c, half, sem):
        ids_u = uid_v[pl.ds(c * LANES, LANES)]
        ids_i = iid_v[pl.ds(c * LANES, LANES)]
        for e in range(HALF):
            k = half * HALF + e
            ucol0 = pl.multiple_of((ids_u[k] // 128) * 128, 128)
            icol0 = pl.multiple_of((ids_i[k] // 128) * 128, 128)
            pltpu.async_copy(uembT_hbm.at[:, pl.ds(ucol0, 128)],
                             ublk_v.at[half, e], sem)
            pltpu.async_copy(iembT_hbm.at[:, pl.ds(icol0, 128)],
                             iblk_v.at[half, e], sem)

    def drain_half(sem):
        for _ in range(2 * HALF):
            pltpu.make_async_copy(
                uembT_hbm.at[:, pl.ds(0, 128)], ublk_v.at[0, 0], sem).wait()

    def compute_half(c, half):
        ids_u = uid_v[pl.ds(c * LANES, LANES)]
        ids_i = iid_v[pl.ds(c * LANES, LANES)]
        for e in range(HALF):
            k = half * HALF + e
            ulane = jnp.full((LANES,), ids_u[k] % 128, jnp.int32)
            ilane = jnp.full((LANES,), ids_i[k] % 128, jnp.int32)
            u = plsc.load_gather(ublk_v.at[half, e], [iota, ulane])
            v = plsc.load_gather(iblk_v.at[half, e], [iota, ilane])
            plsc.store_scatter(pmat_v,
                               [iota, jnp.full((LANES,), k, jnp.int32)],
                               u * v)

    fire_half(0, 0, sem0)
    fire_half(0, 1, sem1)

    def chunk_body(c, carry):
        drain_half(sem0)
        compute_half(c, 0)

        @pl.when(c + 1 < CHUNKS)
        def _():
            fire_half(c + 1, 0, sem0)

        drain_half(sem1)
        compute_half(c, 1)

        @pl.when(c + 1 < CHUNKS)
        def _():
            fire_half(c + 1, 1, sem1)

        dot = pmat_v[0]
        for r in range(1, LANES):
            dot = dot + pmat_v[r]
        out_v[pl.ds(c * LANES, LANES)] = dot
        return carry

    lax.fori_loop(0, CHUNKS, chunk_body, 0)

    pltpu.sync_copy(out_v, out_hbm.at[pl.ds(base, B_PER_W)])


def kernel(user_ids, item_ids, climate_feats, user_emb, item_emb,
           user_bias, item_bias, W_climate, b_climate, global_bias):
    w_bcast = jnp.broadcast_to(W_climate.reshape(N_CLIMATE, 1), (N_CLIMATE, LANES))
    const_bcast = jnp.broadcast_to(b_climate + global_bias, (1, LANES))
    params = jnp.concatenate([w_bcast, const_bcast], axis=0).reshape(5 * LANES)
    uids32 = user_ids.astype(jnp.int32)
    iids32 = item_ids.astype(jnp.int32)
    fm = _fm_kernel(uids32, iids32, user_emb.T, item_emb.T)
    out = _bias_kernel(uids32, iids32, user_bias.T, item_bias.T,
                       climate_feats.reshape(BATCH * N_CLIMATE), params, fm)
    return out.reshape(BATCH, 1)
